# baseline (speedup 1.0000x reference)
import jax
import jax.numpy as jnp
from jax import lax
from jax.experimental import pallas as pl
from jax.experimental.pallas import tpu as pltpu
from jax.experimental.pallas import tpu_sc as plsc

N_NODES = 10000
N_EDGES = 320000
D = 128

NC = 2
NS = 16
NW = NC * NS

EDGES_PER_W = N_EDGES // NW
CHUNK = 40
NCHUNK = EDGES_PER_W // CHUNK
NB = 8
LAG_I = 2
LAG_G = 4
MAIN = (NCHUNK // NB) * NB

ROWS_PER_S = 624
TAIL_ROWS = N_NODES - 15 * ROWS_PER_S


def _sc_agg_body(x_hbm, src_hbm, dst_hbm, zero_hbm, parts_hbm,
                 idxb, rows, acc_sh, isems, gsems, ssems):
    c = lax.axis_index("c")
    s = lax.axis_index("s")
    wid = s * NC + c
    ebase = wid * EDGES_PER_W

    def fire_idx(j, b):
        off = pl.multiple_of(ebase + j * CHUNK, 8)
        pltpu.async_copy(src_hbm.at[pl.ds(off, CHUNK)], idxb.at[b, 0],
                         isems.at[b])
        pltpu.async_copy(dst_hbm.at[pl.ds(off, CHUNK)], idxb.at[b, 1],
                         isems.at[b])

    def wait_idx(b):
        pltpu.make_async_copy(src_hbm.at[pl.ds(0, CHUNK)], idxb.at[b, 0],
                              isems.at[b]).wait()
        pltpu.make_async_copy(dst_hbm.at[pl.ds(0, CHUNK)], idxb.at[b, 1],
                              isems.at[b]).wait()

    def fire_gather(b):
        pltpu.async_copy(x_hbm.at[idxb.at[b, 0]], rows.at[b], gsems.at[b])

    def wait_gather(b):
        pltpu.make_async_copy(x_hbm.at[idxb.at[b, 0]], rows.at[b],
                              gsems.at[b]).wait()

    def fire_scatter(b):
        pltpu.async_copy(rows.at[b], acc_sh.at[idxb.at[b, 1]], ssems.at[b],
                         add=True)

    def wait_scatter(b):
        pltpu.make_async_copy(rows.at[b], acc_sh.at[idxb.at[b, 1]],
                              ssems.at[b]).wait()

    lo = s * ROWS_PER_S
    extra = TAIL_ROWS - ROWS_PER_S

    @pl.when(c == 0)
    def _():
        pltpu.sync_copy(x_hbm.at[pl.ds(lo, ROWS_PER_S)],
                        acc_sh.at[pl.ds(lo, ROWS_PER_S)])

        @pl.when(s == NS - 1)
        def _():
            pltpu.sync_copy(x_hbm.at[pl.ds(16 * ROWS_PER_S, extra)],
                            acc_sh.at[pl.ds(16 * ROWS_PER_S, extra)])

    @pl.when(c != 0)
    def _():
        pltpu.sync_copy(zero_hbm.at[pl.ds(0, ROWS_PER_S)],
                        acc_sh.at[pl.ds(lo, ROWS_PER_S)])

        @pl.when(s == NS - 1)
        def _():
            pltpu.sync_copy(zero_hbm.at[pl.ds(0, extra)],
                            acc_sh.at[pl.ds(16 * ROWS_PER_S, extra)])

    plsc.subcore_barrier()

    @pl.loop(0, MAIN, step=NB)
    def _(g):
        for r in range(NB):
            j = g + r
            b = r
            bg = (r - LAG_I) % NB
            bs = (r - LAG_I - LAG_G) % NB

            @pl.when(j >= NB)
            def _():
                wait_scatter(b)

            fire_idx(j, b)

            @pl.when(j >= LAG_I)
            def _():
                wait_idx(bg)
                fire_gather(bg)

            @pl.when(j >= LAG_I + LAG_G)
            def _():
                wait_gather(bs)
                fire_scatter(bs)

    for j in range(MAIN, NCHUNK + LAG_I + LAG_G):
        b = j % NB
        if j < NCHUNK:
            if j >= NB:
                wait_scatter(b)
            fire_idx(j, b)
        jg = j - LAG_I
        if max(0, MAIN - LAG_I) <= jg < NCHUNK:
            wait_idx(jg % NB)
            fire_gather(jg % NB)
        js = j - LAG_I - LAG_G
        if max(0, MAIN - LAG_I - LAG_G) <= js < NCHUNK:
            wait_gather(js % NB)
            fire_scatter(js % NB)
    for js in range(max(0, NCHUNK - NB), NCHUNK):
        wait_scatter(js % NB)

    plsc.subcore_barrier()

    pltpu.sync_copy(acc_sh.at[pl.ds(lo, ROWS_PER_S)],
                    parts_hbm.at[c, pl.ds(lo, ROWS_PER_S)])

    @pl.when(s == NS - 1)
    def _():
        extra = TAIL_ROWS - ROWS_PER_S
        pltpu.sync_copy(acc_sh.at[pl.ds(16 * ROWS_PER_S, extra)],
                        parts_hbm.at[c, pl.ds(16 * ROWS_PER_S, extra)])


@jax.jit
def _sc_agg(x, src, dst, zero):
    mesh = plsc.VectorSubcoreMesh(core_axis_name="c", subcore_axis_name="s")
    return pl.kernel(
        _sc_agg_body,
        out_type=jax.ShapeDtypeStruct((NC, N_NODES, D), jnp.float32),
        mesh=mesh,
        scratch_types=[
            pltpu.VMEM((NB, 2, CHUNK), jnp.int32),
            pltpu.VMEM((NB, CHUNK, D), jnp.float32),
            pltpu.VMEM_SHARED((N_NODES, D), jnp.float32),
            pltpu.SemaphoreType.DMA((NB,)),
            pltpu.SemaphoreType.DMA((NB,)),
            pltpu.SemaphoreType.DMA((NB,)),
        ],
    )(x, src, dst, zero)


def _tc_mlp_body(parts_ref, wa_ref, ba_ref, wb_ref, bb_ref, o_ref):
    h = parts_ref[0] + parts_ref[1]
    h1 = jnp.dot(h, wa_ref[...], preferred_element_type=jnp.float32,
                 precision=lax.Precision.HIGHEST) + ba_ref[...]
    h1 = jnp.maximum(h1, 0.0)
    o_ref[...] = jnp.dot(h1, wb_ref[...], preferred_element_type=jnp.float32,
                         precision=lax.Precision.HIGHEST) + bb_ref[...]


@jax.jit
def _tc_mlp(parts, wa, ba, wb, bb):
    blk = 5000
    grid = (N_NODES // blk,)
    node_spec = pl.BlockSpec((blk, D), lambda i: (i, 0))
    parts_spec = pl.BlockSpec((NC, blk, D), lambda i: (0, i, 0))
    w_spec = pl.BlockSpec((D, D), lambda i: (0, 0))
    b_spec = pl.BlockSpec((1, D), lambda i: (0, 0))
    return pl.pallas_call(
        _tc_mlp_body,
        grid=grid,
        in_specs=[parts_spec, w_spec, b_spec, w_spec, b_spec],
        out_specs=node_spec,
        out_shape=jax.ShapeDtypeStruct((N_NODES, D), jnp.float32),
    )(parts, wa, ba, wb, bb)


def kernel(x, edge_index, W1a, b1a, W1b, b1b, W2a, b2a, W2b, b2b):
    src = edge_index[0].astype(jnp.int32)
    dst = edge_index[1].astype(jnp.int32)
    zero = jnp.zeros((ROWS_PER_S, D), jnp.float32)

    parts = _sc_agg(x, src, dst, zero)
    h1 = _tc_mlp(parts, W1a, b1a.reshape(1, D), W1b, b1b.reshape(1, D))
    parts2 = _sc_agg(h1, src, dst, zero)
    h2 = _tc_mlp(parts2, W2a, b2a.reshape(1, D), W2b, b2b.reshape(1, D))
    return h2


# speedup vs baseline: 1.1662x; 1.1662x over previous
import jax
import jax.numpy as jnp
from jax import lax
from jax.experimental import pallas as pl
from jax.experimental.pallas import tpu as pltpu
from jax.experimental.pallas import tpu_sc as plsc

N_NODES = 10000
N_EDGES = 320000
D = 128

NC = 2
NS = 16
NW = NC * NS

EDGES_PER_W = N_EDGES // NW
CHUNK = 40
NCHUNK = EDGES_PER_W // CHUNK
NB = 8
LAG_I = 2
LAG_G = 4
MAIN = (NCHUNK // NB) * NB

ROWS_PER_S = 624
TAIL_ROWS = N_NODES - 15 * ROWS_PER_S


def _sc_agg_body(x_hbm, src_hbm, dst_hbm, zero_hbm, parts_hbm,
                 idxb, rows, acc_sh, isems, gsems, ssems):
    c = lax.axis_index("c")
    s = lax.axis_index("s")
    wid = s * NC + c
    ebase = wid * EDGES_PER_W

    def fire_idx(j, b):
        off = pl.multiple_of(ebase + j * CHUNK, 8)
        pltpu.async_copy(src_hbm.at[pl.ds(off, CHUNK)], idxb.at[b, 0],
                         isems.at[b])
        pltpu.async_copy(dst_hbm.at[pl.ds(off, CHUNK)], idxb.at[b, 1],
                         isems.at[b])

    def wait_idx(b):
        pltpu.make_async_copy(src_hbm.at[pl.ds(0, CHUNK)], idxb.at[b, 0],
                              isems.at[b]).wait()
        pltpu.make_async_copy(dst_hbm.at[pl.ds(0, CHUNK)], idxb.at[b, 1],
                              isems.at[b]).wait()

    def fire_gather(b):
        pltpu.async_copy(x_hbm.at[idxb.at[b, 0]], rows.at[b], gsems.at[b])

    def wait_gather(b):
        pltpu.make_async_copy(x_hbm.at[idxb.at[b, 0]], rows.at[b],
                              gsems.at[b]).wait()

    def fire_scatter(b):
        pltpu.async_copy(rows.at[b], acc_sh.at[idxb.at[b, 1]], ssems.at[b],
                         add=True)

    def wait_scatter(b):
        pltpu.make_async_copy(rows.at[b], acc_sh.at[idxb.at[b, 1]],
                              ssems.at[b]).wait()

    lo = s * ROWS_PER_S
    extra = TAIL_ROWS - ROWS_PER_S

    @pl.when(c == 0)
    def _():
        pltpu.sync_copy(x_hbm.at[pl.ds(lo, ROWS_PER_S)],
                        acc_sh.at[pl.ds(lo, ROWS_PER_S)])

        @pl.when(s == NS - 1)
        def _():
            pltpu.sync_copy(x_hbm.at[pl.ds(16 * ROWS_PER_S, extra)],
                            acc_sh.at[pl.ds(16 * ROWS_PER_S, extra)])

    @pl.when(c != 0)
    def _():
        pltpu.sync_copy(zero_hbm.at[pl.ds(0, ROWS_PER_S)],
                        acc_sh.at[pl.ds(lo, ROWS_PER_S)])

        @pl.when(s == NS - 1)
        def _():
            pltpu.sync_copy(zero_hbm.at[pl.ds(0, extra)],
                            acc_sh.at[pl.ds(16 * ROWS_PER_S, extra)])

    plsc.subcore_barrier()

    @pl.loop(0, MAIN, step=NB)
    def _(g):
        for r in range(NB):
            j = g + r
            b = r
            bg = (r - LAG_I) % NB
            bs = (r - LAG_I - LAG_G) % NB

            @pl.when(j >= NB)
            def _():
                wait_scatter(b)

            fire_idx(j, b)

            @pl.when(j >= LAG_I)
            def _():
                wait_idx(bg)
                fire_gather(bg)

            @pl.when(j >= LAG_I + LAG_G)
            def _():
                wait_gather(bs)
                fire_scatter(bs)

    for j in range(MAIN, NCHUNK + LAG_I + LAG_G):
        b = j % NB
        if j < NCHUNK:
            if j >= NB:
                wait_scatter(b)
            fire_idx(j, b)
        jg = j - LAG_I
        if max(0, MAIN - LAG_I) <= jg < NCHUNK:
            wait_idx(jg % NB)
            fire_gather(jg % NB)
        js = j - LAG_I - LAG_G
        if max(0, MAIN - LAG_I - LAG_G) <= js < NCHUNK:
            wait_gather(js % NB)
            fire_scatter(js % NB)
    for js in range(max(0, NCHUNK - NB), NCHUNK):
        wait_scatter(js % NB)

    plsc.subcore_barrier()

    pltpu.sync_copy(acc_sh.at[pl.ds(lo, ROWS_PER_S)],
                    parts_hbm.at[c, pl.ds(lo, ROWS_PER_S)])

    @pl.when(s == NS - 1)
    def _():
        extra = TAIL_ROWS - ROWS_PER_S
        pltpu.sync_copy(acc_sh.at[pl.ds(16 * ROWS_PER_S, extra)],
                        parts_hbm.at[c, pl.ds(16 * ROWS_PER_S, extra)])


@jax.jit
def _sc_agg(x, src, dst, zero):
    mesh = plsc.VectorSubcoreMesh(core_axis_name="c", subcore_axis_name="s")
    return pl.kernel(
        _sc_agg_body,
        out_type=jax.ShapeDtypeStruct((NC, N_NODES, D), jnp.float32),
        mesh=mesh,
        scratch_types=[
            pltpu.VMEM((NB, 2, CHUNK), jnp.int32),
            pltpu.VMEM((NB, CHUNK, D), jnp.float32),
            pltpu.VMEM_SHARED((N_NODES, D), jnp.float32),
            pltpu.SemaphoreType.DMA((NB,)),
            pltpu.SemaphoreType.DMA((NB,)),
            pltpu.SemaphoreType.DMA((NB,)),
        ],
    )(x, src, dst, zero)


def _tc_mlp_body(parts_ref, wa_ref, ba_ref, wb_ref, bb_ref, o_ref):
    h = parts_ref[0] + parts_ref[1]
    h1 = jnp.dot(h, wa_ref[...], preferred_element_type=jnp.float32,
                 precision=lax.Precision.HIGHEST) + ba_ref[...]
    h1 = jnp.maximum(h1, 0.0)
    o_ref[...] = jnp.dot(h1, wb_ref[...], preferred_element_type=jnp.float32,
                         precision=lax.Precision.HIGHEST) + bb_ref[...]


@jax.jit
def _tc_mlp(parts, wa, ba, wb, bb):
    blk = 2000
    grid = (N_NODES // blk,)
    node_spec = pl.BlockSpec((blk, D), lambda i: (i, 0))
    parts_spec = pl.BlockSpec((NC, blk, D), lambda i: (0, i, 0))
    w_spec = pl.BlockSpec((D, D), lambda i: (0, 0))
    b_spec = pl.BlockSpec((1, D), lambda i: (0, 0))
    return pl.pallas_call(
        _tc_mlp_body,
        grid=grid,
        in_specs=[parts_spec, w_spec, b_spec, w_spec, b_spec],
        out_specs=node_spec,
        out_shape=jax.ShapeDtypeStruct((N_NODES, D), jnp.float32),
    )(parts, wa, ba, wb, bb)


def _split_body(ei_ref, s_ref, d_ref):
    s_ref[...] = ei_ref[0]
    d_ref[...] = ei_ref[1]


@jax.jit
def _split_edges(ei):
    return pl.pallas_call(
        _split_body,
        out_shape=[jax.ShapeDtypeStruct((N_EDGES,), jnp.int32),
                   jax.ShapeDtypeStruct((N_EDGES,), jnp.int32)],
    )(ei)


def kernel(x, edge_index, W1a, b1a, W1b, b1b, W2a, b2a, W2b, b2b):
    src, dst = _split_edges(edge_index.astype(jnp.int32))
    zero = jnp.zeros((ROWS_PER_S, D), jnp.float32)

    parts = _sc_agg(x, src, dst, zero)
    h1 = _tc_mlp(parts, W1a, b1a.reshape(1, D), W1b, b1b.reshape(1, D))
    parts2 = _sc_agg(h1, src, dst, zero)
    h2 = _tc_mlp(parts2, W2a, b2a.reshape(1, D), W2b, b2b.reshape(1, D))
    return h2
